# baseline (speedup 1.0000x reference)
import functools

import jax
import jax.numpy as jnp
from jax import lax
from jax.experimental import pallas as pl
from jax.experimental.pallas import tpu as pltpu
from jax.experimental.pallas import tpu_sc as plsc

N = 10000
E = 320000
F = 128
HID = 128
C = 40

NP = 10240
NC = 2
NS = 16
NW = NC * NS
EW = E // NW
K = 80
NCHUNK = EW // K
RT = NP // NS
EV = 80
NB = 25
SB = NCHUNK // NB
R = 3
KD = 80
NCHD = EW // KD

_mesh = plsc.VectorSubcoreMesh(core_axis_name="c", subcore_axis_name="s")


@functools.partial(
    pl.kernel,
    out_type=[
        jax.ShapeDtypeStruct((NP,), jnp.float32),
        jax.ShapeDtypeStruct((NP,), jnp.float32),
    ],
    mesh=_mesh,
    scratch_types=[
        pltpu.VMEM((SB, NB, K), jnp.int32),
        pltpu.VMEM((K,), jnp.float32),
        pltpu.VMEM((RT,), jnp.float32),
        pltpu.VMEM_SHARED((NP,), jnp.float32),
    ],
)
def _sc_degree(dst_hbm, out0, out1, didx_v, ones_v, buf_v, deg_sh):
    c = lax.axis_index("c")
    s = lax.axis_index("s")
    wid = s * NC + c

    pltpu.sync_copy(dst_hbm.at[wid], didx_v)

    for j in range(RT // 16):
        buf_v[pl.ds(j * 16, 16)] = jnp.zeros((16,), jnp.float32)
    for j in range(K // 16):
        ones_v[pl.ds(j * 16, 16)] = jnp.ones((16,), jnp.float32)
    pltpu.sync_copy(buf_v, deg_sh.at[pl.ds(s * RT, RT)])
    plsc.subcore_barrier()

    for b in range(SB):
        def body(i, carry):
            pltpu.sync_copy(ones_v, deg_sh.at[didx_v.at[b, i]], add=True)
            return carry

        lax.fori_loop(0, NB, body, 0, unroll=False)
    plsc.subcore_barrier()

    pltpu.sync_copy(deg_sh.at[pl.ds(s * RT, RT)], buf_v)

    @pl.when(c == 0)
    def _():
        pltpu.sync_copy(buf_v, out0.at[pl.ds(s * RT, RT)])

    @pl.when(c == 1)
    def _():
        pltpu.sync_copy(buf_v, out1.at[pl.ds(s * RT, RT)])


@functools.partial(
    pl.kernel,
    out_type=[
        jax.ShapeDtypeStruct((NP, HID), jnp.float32),
        jax.ShapeDtypeStruct((NP, HID), jnp.float32),
    ],
    mesh=_mesh,
    scratch_types=[
        pltpu.VMEM((NB, K), jnp.int32),
        pltpu.VMEM((NB, K), jnp.int32),
        pltpu.VMEM((K, HID), jnp.float32),
        pltpu.VMEM((K, HID), jnp.float32),
        pltpu.VMEM((K, HID), jnp.float32),
        pltpu.VMEM_SHARED((NP, HID), jnp.float32),
        pltpu.SemaphoreType.DMA,
        pltpu.SemaphoreType.DMA,
        pltpu.SemaphoreType.DMA,
    ],
)
def _sc_scatter(hs_hbm, src_hbm, dst_hbm, out0, out1,
                sidx_v, didx_v, r0_v, r1_v, r2_v, acc_sh, sem0, sem1, sem2):
    c = lax.axis_index("c")
    s = lax.axis_index("s")
    wid = s * NC + c
    rows = (r0_v, r1_v, r2_v)
    sems = (sem0, sem1, sem2)

    z16 = jnp.zeros((16,), jnp.float32)
    for r in range(EV):
        for q in range(HID // 16):
            r0_v[r, pl.ds(q * 16, 16)] = z16
    for j in range(RT // EV):
        pltpu.async_copy(r0_v.at[pl.ds(0, EV), :],
                         acc_sh.at[pl.ds(s * RT + j * EV, EV), :], sem0)
    for j in range(RT // EV):
        pltpu.make_async_copy(r0_v.at[pl.ds(0, EV), :],
                              acc_sh.at[pl.ds(s * RT + j * EV, EV), :],
                              sem0).wait()
    plsc.subcore_barrier()

    def gfire(i, b):
        pltpu.async_copy(hs_hbm.at[sidx_v.at[i]], rows[b], sems[b])

    def sfire(i, b):
        pltpu.async_copy(rows[b], acc_sh.at[didx_v.at[i]], sems[b], add=True)

    def drain(b):
        pltpu.make_async_copy(hs_hbm.at[pl.ds(0, K), :], rows[b], sems[b]).wait()

    NG = NB // R - 1
    EP = R * (NB // R - 1)

    def sblock(blk, carry):
        pltpu.sync_copy(src_hbm.at[wid, blk], sidx_v)
        pltpu.sync_copy(dst_hbm.at[wid, blk], didx_v)
        for b in range(R):
            gfire(b, b)

        def grp(g, c2):
            i0 = R * g
            for b in range(R):
                drain(b)
                sfire(i0 + b, b)
            for b in range(R):
                drain(b)
                gfire(i0 + R + b, b)
            return c2

        lax.fori_loop(0, NG, grp, 0, unroll=False)
        for b in range(R):
            drain(b)
            sfire(EP + b, b)
        drain(0)
        gfire(NB - 1, 0)
        drain(0)
        sfire(NB - 1, 0)
        for b in range(R):
            drain(b)
        return carry

    lax.fori_loop(0, SB, sblock, 0, unroll=False)
    plsc.subcore_barrier()

    def eload(j, b):
        pltpu.async_copy(acc_sh.at[pl.ds(s * RT + j * EV, EV), :],
                         rows[b].at[pl.ds(0, EV), :], sems[b])

    def ewait(b):
        pltpu.make_async_copy(acc_sh.at[pl.ds(0, EV), :],
                              rows[b].at[pl.ds(0, EV), :], sems[b]).wait()

    def estore(j, b):
        @pl.when(c == 0)
        def _():
            pltpu.async_copy(rows[b].at[pl.ds(0, EV), :],
                             out0.at[pl.ds(s * RT + j * EV, EV), :], sems[b])

        @pl.when(c == 1)
        def _():
            pltpu.async_copy(rows[b].at[pl.ds(0, EV), :],
                             out1.at[pl.ds(s * RT + j * EV, EV), :], sems[b])

    NE = RT // EV
    for b in range(R):
        eload(b, b)
    for j in range(NE):
        b = j % R
        ewait(b)
        estore(j, b)
        if j + R < NE:
            ewait(b)
            eload(j + R, b)
    for j in range(NE - R, NE):
        ewait(j % R)


_BR = 1000


def _mm_scale_body(x_ref, w_ref, d0_ref, d1_ref, o_ref):
    dinv = lax.rsqrt(d0_ref[...] + d1_ref[...] + 1.0)
    h = jnp.dot(x_ref[...], w_ref[...], preferred_element_type=jnp.float32)
    o_ref[...] = h * dinv


def _tc_mm_scale(x, w, d0, d1):
    return pl.pallas_call(
        _mm_scale_body,
        grid=(N // _BR,),
        in_specs=[
            pl.BlockSpec((_BR, F), lambda i: (i, 0)),
            pl.BlockSpec((F, HID), lambda i: (0, 0)),
            pl.BlockSpec((_BR, 1), lambda i: (i, 0)),
            pl.BlockSpec((_BR, 1), lambda i: (i, 0)),
        ],
        out_specs=pl.BlockSpec((_BR, HID), lambda i: (i, 0)),
        out_shape=jax.ShapeDtypeStruct((N, HID), jnp.float32),
    )(x, w, d0, d1)


def _fuse_body(a0_ref, a1_ref, hs_ref, d0_ref, d1_ref, b_ref, w_ref, o_ref):
    dinv = lax.rsqrt(d0_ref[...] + d1_ref[...] + 1.0)
    h = (a0_ref[...] + a1_ref[...] + hs_ref[...]) * dinv + b_ref[...]
    h = jnp.maximum(h, 0.0)
    o_ref[...] = jnp.dot(h, w_ref[...], preferred_element_type=jnp.float32) * dinv


def _tc_fuse_mm(a0, a1, hs, d0, d1, b, w):
    return pl.pallas_call(
        _fuse_body,
        grid=(N // _BR,),
        in_specs=[
            pl.BlockSpec((_BR, HID), lambda i: (i, 0)),
            pl.BlockSpec((_BR, HID), lambda i: (i, 0)),
            pl.BlockSpec((_BR, HID), lambda i: (i, 0)),
            pl.BlockSpec((_BR, 1), lambda i: (i, 0)),
            pl.BlockSpec((_BR, 1), lambda i: (i, 0)),
            pl.BlockSpec((1, HID), lambda i: (0, 0)),
            pl.BlockSpec((HID, HID), lambda i: (0, 0)),
        ],
        out_specs=pl.BlockSpec((_BR, HID), lambda i: (i, 0)),
        out_shape=jax.ShapeDtypeStruct((N, HID), jnp.float32),
    )(a0, a1, hs, d0, d1, b, w)


def _head_body(a0_ref, a1_ref, hs_ref, d0_ref, d1_ref, b_ref, wc_ref, bc_ref,
               o_ref):
    dinv = lax.rsqrt(d0_ref[...] + d1_ref[...] + 1.0)
    h = (a0_ref[...] + a1_ref[...] + hs_ref[...]) * dinv + b_ref[...]
    h = jnp.maximum(h, 0.0)
    logits = jnp.dot(h, wc_ref[...], preferred_element_type=jnp.float32)
    logits = logits + bc_ref[...]
    mask = lax.broadcasted_iota(jnp.int32, (1, HID), 1) < C
    neg = jnp.float32(-1e30)
    lm = jnp.where(mask, logits, neg)
    m = jnp.max(lm, axis=1, keepdims=True)
    ex = jnp.where(mask, jnp.exp(logits - m), 0.0)
    lse = jnp.log(jnp.sum(ex, axis=1, keepdims=True))
    o_ref[...] = (logits - m - lse)[:, :C]


def _tc_head(a0, a1, hs, d0, d1, b, wc, bc):
    return pl.pallas_call(
        _head_body,
        grid=(N // _BR,),
        in_specs=[
            pl.BlockSpec((_BR, HID), lambda i: (i, 0)),
            pl.BlockSpec((_BR, HID), lambda i: (i, 0)),
            pl.BlockSpec((_BR, HID), lambda i: (i, 0)),
            pl.BlockSpec((_BR, 1), lambda i: (i, 0)),
            pl.BlockSpec((_BR, 1), lambda i: (i, 0)),
            pl.BlockSpec((1, HID), lambda i: (0, 0)),
            pl.BlockSpec((HID, HID), lambda i: (0, 0)),
            pl.BlockSpec((1, HID), lambda i: (0, 0)),
        ],
        out_specs=pl.BlockSpec((_BR, C), lambda i: (i, 0)),
        out_shape=jax.ShapeDtypeStruct((N, C), jnp.float32),
    )(a0, a1, hs, d0, d1, b, wc, bc)


def kernel(x, edge_index, W1, b1, W2, b2, Wc, bc):
    src4 = edge_index[0].reshape(NW, SB, NB, K)
    dst4 = edge_index[1].reshape(NW, SB, NB, K)

    deg0, deg1 = _sc_degree(dst4)
    d0 = deg0.reshape(NP, 1)
    d1 = deg1.reshape(NP, 1)

    b1r = b1.reshape(1, HID)
    b2r = b2.reshape(1, HID)
    wc_pad = jnp.zeros((HID, HID), jnp.float32).at[:, :C].set(Wc)
    bc_pad = jnp.zeros((1, HID), jnp.float32).at[0, :C].set(bc)

    h1s = _tc_mm_scale(x, W1, d0, d1)
    a0, a1 = _sc_scatter(h1s, src4, dst4)
    h2s = _tc_fuse_mm(a0, a1, h1s, d0, d1, b1r, W2)
    c0, c1 = _sc_scatter(h2s, src4, dst4)
    return _tc_head(c0, c1, h2s, d0, d1, b2r, wc_pad, bc_pad)


# speedup vs baseline: 1.0649x; 1.0649x over previous
import functools

import jax
import jax.numpy as jnp
from jax import lax
from jax.experimental import pallas as pl
from jax.experimental.pallas import tpu as pltpu
from jax.experimental.pallas import tpu_sc as plsc

N = 10000
E = 320000
F = 128
HID = 128
C = 40

NP = 10240
NC = 2
NS = 16
NW = NC * NS
EW = E // NW
K = 80
NCHUNK = EW // K
RT = NP // NS
EV = 80
NB = 25
SB = NCHUNK // NB
R = 4
KD = 80
NCHD = EW // KD

_mesh = plsc.VectorSubcoreMesh(core_axis_name="c", subcore_axis_name="s")


@functools.partial(
    pl.kernel,
    out_type=[
        jax.ShapeDtypeStruct((NP,), jnp.float32),
        jax.ShapeDtypeStruct((NP,), jnp.float32),
    ],
    mesh=_mesh,
    scratch_types=[
        pltpu.VMEM((SB, NB, K), jnp.int32),
        pltpu.VMEM((K,), jnp.float32),
        pltpu.VMEM((RT,), jnp.float32),
        pltpu.VMEM_SHARED((NP,), jnp.float32),
    ],
)
def _sc_degree(dst_hbm, out0, out1, didx_v, ones_v, buf_v, deg_sh):
    c = lax.axis_index("c")
    s = lax.axis_index("s")
    wid = s * NC + c

    pltpu.sync_copy(dst_hbm.at[wid], didx_v)

    for j in range(RT // 16):
        buf_v[pl.ds(j * 16, 16)] = jnp.zeros((16,), jnp.float32)
    for j in range(K // 16):
        ones_v[pl.ds(j * 16, 16)] = jnp.ones((16,), jnp.float32)
    pltpu.sync_copy(buf_v, deg_sh.at[pl.ds(s * RT, RT)])
    plsc.subcore_barrier()

    for b in range(SB):
        def body(i, carry):
            pltpu.sync_copy(ones_v, deg_sh.at[didx_v.at[b, i]], add=True)
            return carry

        lax.fori_loop(0, NB, body, 0, unroll=False)
    plsc.subcore_barrier()

    pltpu.sync_copy(deg_sh.at[pl.ds(s * RT, RT)], buf_v)

    @pl.when(c == 0)
    def _():
        pltpu.sync_copy(buf_v, out0.at[pl.ds(s * RT, RT)])

    @pl.when(c == 1)
    def _():
        pltpu.sync_copy(buf_v, out1.at[pl.ds(s * RT, RT)])


@functools.partial(
    pl.kernel,
    out_type=[
        jax.ShapeDtypeStruct((NP, HID), jnp.float32),
        jax.ShapeDtypeStruct((NP, HID), jnp.float32),
    ],
    mesh=_mesh,
    scratch_types=[
        pltpu.VMEM((NB, K), jnp.int32),
        pltpu.VMEM((NB, K), jnp.int32),
        pltpu.VMEM((K, HID), jnp.float32),
        pltpu.VMEM((K, HID), jnp.float32),
        pltpu.VMEM((K, HID), jnp.float32),
        pltpu.VMEM((K, HID), jnp.float32),
        pltpu.VMEM_SHARED((NP, HID), jnp.float32),
        pltpu.SemaphoreType.DMA,
        pltpu.SemaphoreType.DMA,
        pltpu.SemaphoreType.DMA,
        pltpu.SemaphoreType.DMA,
    ],
)
def _sc_scatter(hs_hbm, src_hbm, dst_hbm, out0, out1,
                sidx_v, didx_v, r0_v, r1_v, r2_v, r3_v, acc_sh,
                sem0, sem1, sem2, sem3):
    c = lax.axis_index("c")
    s = lax.axis_index("s")
    wid = s * NC + c
    rows = (r0_v, r1_v, r2_v, r3_v)
    sems = (sem0, sem1, sem2, sem3)

    z16 = jnp.zeros((16,), jnp.float32)
    for r in range(EV):
        for q in range(HID // 16):
            r0_v[r, pl.ds(q * 16, 16)] = z16
    for j in range(RT // EV):
        pltpu.async_copy(r0_v.at[pl.ds(0, EV), :],
                         acc_sh.at[pl.ds(s * RT + j * EV, EV), :], sem0)
    for j in range(RT // EV):
        pltpu.make_async_copy(r0_v.at[pl.ds(0, EV), :],
                              acc_sh.at[pl.ds(s * RT + j * EV, EV), :],
                              sem0).wait()
    plsc.subcore_barrier()

    def gfire(i, b):
        pltpu.async_copy(hs_hbm.at[sidx_v.at[i]], rows[b], sems[b])

    def sfire(i, b):
        pltpu.async_copy(rows[b], acc_sh.at[didx_v.at[i]], sems[b], add=True)

    def drain(b):
        pltpu.make_async_copy(hs_hbm.at[pl.ds(0, K), :], rows[b], sems[b]).wait()

    NG = NB // R - 1
    EP = R * (NB // R - 1)

    def sblock(blk, carry):
        pltpu.sync_copy(src_hbm.at[wid, blk], sidx_v)
        pltpu.sync_copy(dst_hbm.at[wid, blk], didx_v)
        for b in range(R):
            gfire(b, b)

        def grp(g, c2):
            i0 = R * g
            for b in range(R):
                drain(b)
                sfire(i0 + b, b)
            for b in range(R):
                drain(b)
                gfire(i0 + R + b, b)
            return c2

        lax.fori_loop(0, NG, grp, 0, unroll=False)
        for b in range(R):
            drain(b)
            sfire(EP + b, b)
        drain(0)
        gfire(NB - 1, 0)
        drain(0)
        sfire(NB - 1, 0)
        for b in range(R):
            drain(b)
        return carry

    lax.fori_loop(0, SB, sblock, 0, unroll=False)
    plsc.subcore_barrier()

    def eload(j, b):
        pltpu.async_copy(acc_sh.at[pl.ds(s * RT + j * EV, EV), :],
                         rows[b].at[pl.ds(0, EV), :], sems[b])

    def ewait(b):
        pltpu.make_async_copy(acc_sh.at[pl.ds(0, EV), :],
                              rows[b].at[pl.ds(0, EV), :], sems[b]).wait()

    def estore(j, b):
        @pl.when(c == 0)
        def _():
            pltpu.async_copy(rows[b].at[pl.ds(0, EV), :],
                             out0.at[pl.ds(s * RT + j * EV, EV), :], sems[b])

        @pl.when(c == 1)
        def _():
            pltpu.async_copy(rows[b].at[pl.ds(0, EV), :],
                             out1.at[pl.ds(s * RT + j * EV, EV), :], sems[b])

    NE = RT // EV
    for b in range(R):
        eload(b, b)
    for j in range(NE):
        b = j % R
        ewait(b)
        estore(j, b)
        if j + R < NE:
            ewait(b)
            eload(j + R, b)
    for j in range(NE - R, NE):
        ewait(j % R)


_BR = 1000


def _mm_scale_body(x_ref, w_ref, d0_ref, d1_ref, o_ref):
    dinv = lax.rsqrt(d0_ref[...] + d1_ref[...] + 1.0)
    h = jnp.dot(x_ref[...], w_ref[...], preferred_element_type=jnp.float32)
    o_ref[...] = h * dinv


def _tc_mm_scale(x, w, d0, d1):
    return pl.pallas_call(
        _mm_scale_body,
        grid=(N // _BR,),
        in_specs=[
            pl.BlockSpec((_BR, F), lambda i: (i, 0)),
            pl.BlockSpec((F, HID), lambda i: (0, 0)),
            pl.BlockSpec((_BR, 1), lambda i: (i, 0)),
            pl.BlockSpec((_BR, 1), lambda i: (i, 0)),
        ],
        out_specs=pl.BlockSpec((_BR, HID), lambda i: (i, 0)),
        out_shape=jax.ShapeDtypeStruct((N, HID), jnp.float32),
    )(x, w, d0, d1)


def _fuse_body(a0_ref, a1_ref, hs_ref, d0_ref, d1_ref, b_ref, w_ref, o_ref):
    dinv = lax.rsqrt(d0_ref[...] + d1_ref[...] + 1.0)
    h = (a0_ref[...] + a1_ref[...] + hs_ref[...]) * dinv + b_ref[...]
    h = jnp.maximum(h, 0.0)
    o_ref[...] = jnp.dot(h, w_ref[...], preferred_element_type=jnp.float32) * dinv


def _tc_fuse_mm(a0, a1, hs, d0, d1, b, w):
    return pl.pallas_call(
        _fuse_body,
        grid=(N // _BR,),
        in_specs=[
            pl.BlockSpec((_BR, HID), lambda i: (i, 0)),
            pl.BlockSpec((_BR, HID), lambda i: (i, 0)),
            pl.BlockSpec((_BR, HID), lambda i: (i, 0)),
            pl.BlockSpec((_BR, 1), lambda i: (i, 0)),
            pl.BlockSpec((_BR, 1), lambda i: (i, 0)),
            pl.BlockSpec((1, HID), lambda i: (0, 0)),
            pl.BlockSpec((HID, HID), lambda i: (0, 0)),
        ],
        out_specs=pl.BlockSpec((_BR, HID), lambda i: (i, 0)),
        out_shape=jax.ShapeDtypeStruct((N, HID), jnp.float32),
    )(a0, a1, hs, d0, d1, b, w)


def _head_body(a0_ref, a1_ref, hs_ref, d0_ref, d1_ref, b_ref, wc_ref, bc_ref,
               o_ref):
    dinv = lax.rsqrt(d0_ref[...] + d1_ref[...] + 1.0)
    h = (a0_ref[...] + a1_ref[...] + hs_ref[...]) * dinv + b_ref[...]
    h = jnp.maximum(h, 0.0)
    logits = jnp.dot(h, wc_ref[...], preferred_element_type=jnp.float32)
    logits = logits + bc_ref[...]
    mask = lax.broadcasted_iota(jnp.int32, (1, HID), 1) < C
    neg = jnp.float32(-1e30)
    lm = jnp.where(mask, logits, neg)
    m = jnp.max(lm, axis=1, keepdims=True)
    ex = jnp.where(mask, jnp.exp(logits - m), 0.0)
    lse = jnp.log(jnp.sum(ex, axis=1, keepdims=True))
    o_ref[...] = (logits - m - lse)[:, :C]


def _tc_head(a0, a1, hs, d0, d1, b, wc, bc):
    return pl.pallas_call(
        _head_body,
        grid=(N // _BR,),
        in_specs=[
            pl.BlockSpec((_BR, HID), lambda i: (i, 0)),
            pl.BlockSpec((_BR, HID), lambda i: (i, 0)),
            pl.BlockSpec((_BR, HID), lambda i: (i, 0)),
            pl.BlockSpec((_BR, 1), lambda i: (i, 0)),
            pl.BlockSpec((_BR, 1), lambda i: (i, 0)),
            pl.BlockSpec((1, HID), lambda i: (0, 0)),
            pl.BlockSpec((HID, HID), lambda i: (0, 0)),
            pl.BlockSpec((1, HID), lambda i: (0, 0)),
        ],
        out_specs=pl.BlockSpec((_BR, C), lambda i: (i, 0)),
        out_shape=jax.ShapeDtypeStruct((N, C), jnp.float32),
    )(a0, a1, hs, d0, d1, b, wc, bc)


def kernel(x, edge_index, W1, b1, W2, b2, Wc, bc):
    src4 = edge_index[0].reshape(NW, SB, NB, K)
    dst4 = edge_index[1].reshape(NW, SB, NB, K)

    deg0, deg1 = _sc_degree(dst4)
    d0 = deg0.reshape(NP, 1)
    d1 = deg1.reshape(NP, 1)

    b1r = b1.reshape(1, HID)
    b2r = b2.reshape(1, HID)
    wc_pad = jnp.zeros((HID, HID), jnp.float32).at[:, :C].set(Wc)
    bc_pad = jnp.zeros((1, HID), jnp.float32).at[0, :C].set(bc)

    h1s = _tc_mm_scale(x, W1, d0, d1)
    a0, a1 = _sc_scatter(h1s, src4, dst4)
    h2s = _tc_fuse_mm(a0, a1, h1s, d0, d1, b1r, W2)
    c0, c1 = _sc_scatter(h2s, src4, dst4)
    return _tc_head(c0, c1, h2s, d0, d1, b2r, wc_pad, bc_pad)
